# baseline (speedup 1.0000x reference)
import functools
import jax
import jax.numpy as jnp
from jax import lax
from jax.experimental import pallas as pl
from jax.experimental.pallas import tpu as pltpu
from jax.experimental.pallas import tpu_sc as plsc

N = 10000
E = 320000
D = 128
DE = 16
G = 64
H1 = 128
MLP_DIM = 256
NCLS = 4

NC = 2
NS = 16
NW = NC * NS
CHUNK = 128
NCH = 80
EPW = NCH * CHUNK
EPAD = EPW * NW
NACC = 10112
RPS = NACC // NS
ZOFF = tuple(range(0, RPS - CHUNK, CHUNK)) + (RPS - CHUNK,)

RB = 1000
NBLK = N // RB

_BN = float(1.0 / (1.0 + 1e-5) ** 0.5)


def _dot(a, b):
  return lax.dot_general(a, b, (((1,), (0,)), ((), ())),
                         precision=lax.Precision.HIGHEST,
                         preferred_element_type=jnp.float32)


def _dot_t(a, b):
  return lax.dot_general(a, b, (((0,), (0,)), ((), ())),
                         precision=lax.Precision.HIGHEST,
                         preferred_element_type=jnp.float32)


def _sc_mesh():
  return plsc.VectorSubcoreMesh(core_axis_name="c", subcore_axis_name="s")


def _make_sc_edge_agg():
  out_type = jax.ShapeDtypeStruct((NC, NACC, D), jnp.float32)
  scratch = [
      pltpu.VMEM((CHUNK,), jnp.int32),
      pltpu.VMEM((CHUNK,), jnp.int32),
      pltpu.VMEM((CHUNK,), jnp.int32),
      pltpu.VMEM((CHUNK,), jnp.int32),
      pltpu.VMEM((CHUNK, D), jnp.float32),
      pltpu.VMEM_SHARED((NACC, D), jnp.float32),
      pltpu.SemaphoreType.DMA,
      pltpu.SemaphoreType.DMA,
      pltpu.SemaphoreType.DMA,
      pltpu.SemaphoreType.DMA,
  ]

  def body(table_hbm, src_hbm, dst_hbm, a_out, srcv0, srcv1, dstv0, dstv1,
           rows, a_sh, isem0, isem1, dsem0, dsem1):
    c = lax.axis_index("c")
    s = lax.axis_index("s")
    w = c * NS + s

    zrow = jnp.zeros((1, 16), jnp.float32)

    @pl.loop(0, CHUNK)
    def _(r):
      for cc in range(D // 16):
        rows.at[pl.ds(r, 1), pl.ds(cc * 16, 16)][...] = zrow

    for z in ZOFF:
      pltpu.sync_copy(rows, a_sh.at[pl.ds(s * RPS + z, CHUNK)])

    pltpu.async_copy(src_hbm.at[w, 0], srcv0, isem0)
    pltpu.async_copy(dst_hbm.at[w, 0], dstv0, dsem0)
    pltpu.async_copy(src_hbm.at[w, 1], srcv1, isem1)
    pltpu.async_copy(dst_hbm.at[w, 1], dstv1, dsem1)

    plsc.subcore_barrier()

    @pl.loop(0, NCH // 2)
    def _(jj):
      j = jj * 2
      pltpu.make_async_copy(src_hbm.at[w, j], srcv0, isem0).wait()
      pltpu.sync_copy(table_hbm.at[srcv0], rows)

      @pl.when(jj < NCH // 2 - 1)
      def _():
        pltpu.async_copy(src_hbm.at[w, j + 2], srcv0, isem0)

      pltpu.make_async_copy(dst_hbm.at[w, j], dstv0, dsem0).wait()
      pltpu.sync_copy(rows, a_sh.at[dstv0], add=True)

      @pl.when(jj < NCH // 2 - 1)
      def _():
        pltpu.async_copy(dst_hbm.at[w, j + 2], dstv0, dsem0)

      pltpu.make_async_copy(src_hbm.at[w, j + 1], srcv1, isem1).wait()
      pltpu.sync_copy(table_hbm.at[srcv1], rows)

      @pl.when(jj < NCH // 2 - 1)
      def _():
        pltpu.async_copy(src_hbm.at[w, j + 3], srcv1, isem1)

      pltpu.make_async_copy(dst_hbm.at[w, j + 1], dstv1, dsem1).wait()
      pltpu.sync_copy(rows, a_sh.at[dstv1], add=True)

      @pl.when(jj < NCH // 2 - 1)
      def _():
        pltpu.async_copy(dst_hbm.at[w, j + 3], dstv1, dsem1)

    plsc.subcore_barrier()

    pltpu.sync_copy(a_sh.at[pl.ds(s * RPS, RPS)],
                    a_out.at[c, pl.ds(s * RPS, RPS)])

  return pl.kernel(body, mesh=_sc_mesh(), out_type=out_type,
                   scratch_types=scratch)


def _make_sc_edge_sh():
  out_type = jax.ShapeDtypeStruct((NC, NACC, D), jnp.float32)
  scratch = [
      pltpu.VMEM((CHUNK,), jnp.int32),
      pltpu.VMEM((CHUNK,), jnp.int32),
      pltpu.VMEM((CHUNK, DE), jnp.float32),
      pltpu.VMEM((CHUNK, DE), jnp.float32),
      pltpu.VMEM((CHUNK, D), jnp.float32),
      pltpu.VMEM_SHARED((NACC, D), jnp.float32),
      pltpu.SemaphoreType.DMA,
      pltpu.SemaphoreType.DMA,
      pltpu.SemaphoreType.DMA,
      pltpu.SemaphoreType.DMA,
  ]

  def body(dst_hbm, ea_hbm, sh_out, dstv0, dstv1, ea0, ea1, wbuf, sh_acc,
           esem0, esem1, dsem0, dsem1):
    c = lax.axis_index("c")
    s = lax.axis_index("s")
    w = c * NS + s

    zrow = jnp.zeros((1, 16), jnp.float32)
    e0 = jnp.where(lax.iota(jnp.int32, 16) < 1,
                   jnp.float32(1.0), jnp.float32(0.0)).reshape(1, 16)

    @pl.loop(0, CHUNK)
    def _(r):
      for cc in range(D // 16):
        wbuf.at[pl.ds(r, 1), pl.ds(cc * 16, 16)][...] = zrow

    for z in ZOFF:
      pltpu.sync_copy(wbuf, sh_acc.at[pl.ds(s * RPS + z, CHUNK)])

    @pl.loop(0, CHUNK)
    def _(r):
      wbuf.at[pl.ds(r, 1), pl.ds(DE, 16)][...] = e0

    pltpu.async_copy(dst_hbm.at[w, 0], dstv0, dsem0)
    pltpu.async_copy(dst_hbm.at[w, 1], dstv1, dsem1)

    plsc.subcore_barrier()

    pltpu.async_copy(ea_hbm.at[w, pl.ds(0, CHUNK)], ea0, esem0)
    pltpu.async_copy(ea_hbm.at[w, pl.ds(CHUNK, CHUNK)], ea1, esem1)

    @pl.loop(0, NCH // 2)
    def _(jj):
      j = jj * 2
      pltpu.make_async_copy(ea_hbm.at[w, pl.ds(0, CHUNK)], ea0, esem0).wait()

      @pl.loop(0, CHUNK)
      def _(r):
        wbuf.at[pl.ds(r, 1), pl.ds(0, DE)][...] = ea0.at[pl.ds(r, 1), :][...]

      @pl.when(jj < NCH // 2 - 1)
      def _():
        pltpu.async_copy(ea_hbm.at[w, pl.ds((j + 2) * CHUNK, CHUNK)], ea0,
                         esem0)

      pltpu.make_async_copy(dst_hbm.at[w, j], dstv0, dsem0).wait()
      pltpu.sync_copy(wbuf, sh_acc.at[dstv0], add=True)

      @pl.when(jj < NCH // 2 - 1)
      def _():
        pltpu.async_copy(dst_hbm.at[w, j + 2], dstv0, dsem0)

      pltpu.make_async_copy(ea_hbm.at[w, pl.ds(0, CHUNK)], ea1, esem1).wait()

      @pl.loop(0, CHUNK)
      def _(r):
        wbuf.at[pl.ds(r, 1), pl.ds(0, DE)][...] = ea1.at[pl.ds(r, 1), :][...]

      @pl.when(jj < NCH // 2 - 1)
      def _():
        pltpu.async_copy(ea_hbm.at[w, pl.ds((j + 3) * CHUNK, CHUNK)], ea1,
                         esem1)

      pltpu.make_async_copy(dst_hbm.at[w, j + 1], dstv1, dsem1).wait()
      pltpu.sync_copy(wbuf, sh_acc.at[dstv1], add=True)

      @pl.when(jj < NCH // 2 - 1)
      def _():
        pltpu.async_copy(dst_hbm.at[w, j + 3], dstv1, dsem1)

    plsc.subcore_barrier()

    pltpu.sync_copy(sh_acc.at[pl.ds(s * RPS, RPS)],
                    sh_out.at[c, pl.ds(s * RPS, RPS)])

  return pl.kernel(body, mesh=_sc_mesh(), out_type=out_type,
                   scratch_types=scratch)


def _layer_block(x_ref, a_ref, sh_ref, w_ref, b_ref, g_ref, bt_ref):
  xb = x_ref[...]
  a = a_ref[...]
  sh = sh_ref[...]
  ab = a[0] + a[1] + xb
  sb = sh[0, :, 0:DE] + sh[1, :, 0:DE] + 1.0
  cnt = sh[0, :, DE:DE + 1] + sh[1, :, DE:DE + 1] + 1.0
  wi = w_ref[0:D, :]
  wj = w_ref[D:2 * D, :]
  we = w_ref[2 * D:2 * D + DE, :]
  agg = cnt * (_dot(xb, wi) + b_ref[...]) + _dot(ab, wj) + _dot(sb, we)
  h = jnp.maximum(agg, 0.0)
  h = h * (g_ref[...] * _BN) + bt_ref[...]
  return jnp.maximum(h, 0.0)


def _tc_layer1(x, a_part, sh_part, W, b, g, bt):
  def kern(x_ref, a_ref, sh_ref, w_ref, b_ref, g_ref, bt_ref, o_ref):
    o_ref[...] = _layer_block(x_ref, a_ref, sh_ref, w_ref, b_ref,
                              g_ref, bt_ref)

  return pl.pallas_call(
      kern,
      grid=(NBLK,),
      in_specs=[
          pl.BlockSpec((RB, D), lambda i: (i, 0)),
          pl.BlockSpec((NC, RB, D), lambda i: (0, i, 0)),
          pl.BlockSpec((NC, RB, D), lambda i: (0, i, 0)),
          pl.BlockSpec((2 * D + DE, H1), lambda i: (0, 0)),
          pl.BlockSpec((1, H1), lambda i: (0, 0)),
          pl.BlockSpec((1, H1), lambda i: (0, 0)),
          pl.BlockSpec((1, H1), lambda i: (0, 0)),
      ],
      out_specs=pl.BlockSpec((RB, H1), lambda i: (i, 0)),
      out_shape=jax.ShapeDtypeStruct((N, H1), jnp.float32),
  )(x, a_part, sh_part, W, b, g, bt)


def _tc_layer2(h1, a_part, sh_part, batch3, neighbor, W, b, g, bt,
               wf1a, wf1b, wf1c, bf1, wf2, bf2):
  def kern(x_ref, a_ref, sh_ref, batch_ref, nb_ref, w_ref, b_ref,
           g_ref, bt_ref, wf1a_ref, wf1b_ref, wf1c_ref, bf1_ref, wf2_ref,
           bf2_ref, o_ref, pool_scr, cnt_scr):
    i = pl.program_id(0)

    @pl.when(i == 0)
    def _():
      pool_scr[...] = jnp.zeros_like(pool_scr)
      cnt_scr[...] = jnp.zeros_like(cnt_scr)

    h2 = _layer_block(x_ref, a_ref, sh_ref, w_ref, b_ref, g_ref, bt_ref)
    bvec = batch_ref[0, 0, :]
    onehot = (bvec[:, None] ==
              lax.broadcasted_iota(jnp.int32, (RB, G), 1)).astype(jnp.float32)
    pool_scr[...] += _dot_t(onehot, h2)
    cnt_scr[...] += _dot_t(onehot, jnp.ones((RB, 8), jnp.float32))

    @pl.when(i == NBLK - 1)
    def _():
      pooled = pool_scr[...]
      counts = cnt_scr[...][:, 0:1] * (1.0 / 40.0)
      hid = (_dot(pooled, wf1a_ref[...]) + counts * wf1b_ref[...] +
             _dot(nb_ref[...], wf1c_ref[...]) + bf1_ref[...])
      hid = jnp.maximum(hid, 0.0)
      o_ref[...] = _dot(hid, wf2_ref[...]) + bf2_ref[...]

  return pl.pallas_call(
      kern,
      grid=(NBLK,),
      in_specs=[
          pl.BlockSpec((RB, D), lambda i: (i, 0)),
          pl.BlockSpec((NC, RB, D), lambda i: (0, i, 0)),
          pl.BlockSpec((NC, RB, D), lambda i: (0, i, 0)),
          pl.BlockSpec((1, 1, RB), lambda i: (i, 0, 0)),
          pl.BlockSpec((G, D), lambda i: (0, 0)),
          pl.BlockSpec((2 * D + DE, H1), lambda i: (0, 0)),
          pl.BlockSpec((1, H1), lambda i: (0, 0)),
          pl.BlockSpec((1, H1), lambda i: (0, 0)),
          pl.BlockSpec((1, H1), lambda i: (0, 0)),
          pl.BlockSpec((D, MLP_DIM), lambda i: (0, 0)),
          pl.BlockSpec((1, MLP_DIM), lambda i: (0, 0)),
          pl.BlockSpec((D, MLP_DIM), lambda i: (0, 0)),
          pl.BlockSpec((1, MLP_DIM), lambda i: (0, 0)),
          pl.BlockSpec((MLP_DIM, NCLS), lambda i: (0, 0)),
          pl.BlockSpec((1, NCLS), lambda i: (0, 0)),
      ],
      out_specs=pl.BlockSpec((G, NCLS), lambda i: (0, 0)),
      out_shape=jax.ShapeDtypeStruct((G, NCLS), jnp.float32),
      scratch_shapes=[pltpu.VMEM((G, D), jnp.float32),
                      pltpu.VMEM((G, 8), jnp.float32)],
  )(h1, a_part, sh_part, batch3, neighbor, W, b, g, bt,
    wf1a, wf1b, wf1c, bf1, wf2, bf2)


def kernel(x, edge_index, edge_attr, batch, neighbor, W1, b1, g1, bt1,
           W2, b2, g2, bt2, Wf1, bf1, Wf2, bf2):
  pad = EPAD - E
  src = jnp.concatenate([edge_index[0].astype(jnp.int32),
                         jnp.zeros((pad,), jnp.int32)]).reshape(NW, NCH, CHUNK)
  dst = jnp.concatenate([edge_index[1].astype(jnp.int32),
                         jnp.full((pad,), N, jnp.int32)]).reshape(NW, NCH,
                                                                 CHUNK)
  ea = jnp.concatenate([edge_attr, jnp.zeros((pad, DE), jnp.float32)],
                       axis=0).reshape(NW, EPW, DE)

  a1 = _make_sc_edge_agg()(x, src, dst)
  sh = _make_sc_edge_sh()(dst, ea)
  h1 = _tc_layer1(x, a1, sh, W1, b1.reshape(1, -1),
                  g1.reshape(1, -1), bt1.reshape(1, -1))
  a2 = _make_sc_edge_agg()(h1, src, dst)
  batch3 = batch.astype(jnp.int32).reshape(NBLK, 1, RB)
  out = _tc_layer2(h1, a2, sh, batch3, neighbor, W2,
                   b2.reshape(1, -1), g2.reshape(1, -1), bt2.reshape(1, -1),
                   Wf1[0:D], Wf1[D:D + 1], Wf1[D + 1:], bf1.reshape(1, -1),
                   Wf2, bf2.reshape(1, -1))
  return out


# speedup vs baseline: 1.0113x; 1.0113x over previous
import functools
import jax
import jax.numpy as jnp
from jax import lax
from jax.experimental import pallas as pl
from jax.experimental.pallas import tpu as pltpu
from jax.experimental.pallas import tpu_sc as plsc

N = 10000
E = 320000
D = 128
DE = 16
G = 64
H1 = 128
MLP_DIM = 256
NCLS = 4

NC = 2
NS = 16
NW = NC * NS
CHA = 128
NCHA = 80
CHS = 64
NCHS = 160
EPW = NCHA * CHA
EPAD = EPW * NW
NACC = 10112
RPS = NACC // NS
ZOFFA = tuple(range(0, RPS - CHA, CHA)) + (RPS - CHA,)
ZOFFS = tuple(range(0, RPS - CHS, CHS)) + (RPS - CHS,)

RB = 1000
NBLK = N // RB

_BN = float(1.0 / (1.0 + 1e-5) ** 0.5)


def _dot(a, b):
  return lax.dot_general(a, b, (((1,), (0,)), ((), ())),
                         precision=lax.Precision.HIGHEST,
                         preferred_element_type=jnp.float32)


def _dot_t(a, b):
  return lax.dot_general(a, b, (((0,), (0,)), ((), ())),
                         precision=lax.Precision.HIGHEST,
                         preferred_element_type=jnp.float32)


def _sc_mesh():
  return plsc.VectorSubcoreMesh(core_axis_name="c", subcore_axis_name="s")


def _make_sc_edge_agg():
  out_type = jax.ShapeDtypeStruct((NC, NACC, D), jnp.float32)
  scratch = [
      pltpu.VMEM((CHA,), jnp.int32),
      pltpu.VMEM((CHA,), jnp.int32),
      pltpu.VMEM((CHA, D), jnp.float32),
      pltpu.VMEM_SHARED((NACC, D), jnp.float32),
      pltpu.SemaphoreType.DMA,
  ]

  def body(table_hbm, src_hbm, dst_hbm, a_out, srcv, dstv, rows, a_sh, sem):
    c = lax.axis_index("c")
    s = lax.axis_index("s")
    w = c * NS + s

    zrow = jnp.zeros((1, 16), jnp.float32)

    @pl.loop(0, CHA)
    def _(r):
      for cc in range(D // 16):
        rows.at[pl.ds(r, 1), pl.ds(cc * 16, 16)][...] = zrow

    for z in ZOFFA:
      pltpu.sync_copy(rows, a_sh.at[pl.ds(s * RPS + z, CHA)])

    plsc.subcore_barrier()

    base = w * EPW

    @pl.loop(0, NCHA)
    def _(j):
      off = base + j * CHA
      pltpu.sync_copy(src_hbm.at[pl.ds(off, CHA)], srcv)
      pltpu.sync_copy(dst_hbm.at[pl.ds(off, CHA)], dstv)
      pltpu.async_copy(table_hbm.at[srcv], rows, sem).wait()
      pltpu.sync_copy(rows, a_sh.at[dstv], add=True)

    plsc.subcore_barrier()

    pltpu.sync_copy(a_sh.at[pl.ds(s * RPS, RPS)],
                    a_out.at[c, pl.ds(s * RPS, RPS)])

  return pl.kernel(body, mesh=_sc_mesh(), out_type=out_type,
                   scratch_types=scratch)


def _make_sc_edge_sh():
  out_type = jax.ShapeDtypeStruct((NC, NACC, D), jnp.float32)
  scratch = [
      pltpu.VMEM((CHS,), jnp.int32),
      pltpu.VMEM((CHS,), jnp.int32),
      pltpu.VMEM((CHS, DE), jnp.float32),
      pltpu.VMEM((CHS, DE), jnp.float32),
      pltpu.VMEM((CHS, D), jnp.float32),
      pltpu.VMEM_SHARED((NACC, D), jnp.float32),
      pltpu.SemaphoreType.DMA,
      pltpu.SemaphoreType.DMA,
      pltpu.SemaphoreType.DMA,
      pltpu.SemaphoreType.DMA,
  ]

  def body(dst_hbm, ea_hbm, sh_out, dstv0, dstv1, ea0, ea1, wbuf, sh_acc,
           esem0, esem1, dsem0, dsem1):
    c = lax.axis_index("c")
    s = lax.axis_index("s")
    w = c * NS + s

    zrow = jnp.zeros((1, 16), jnp.float32)
    e0 = jnp.where(lax.iota(jnp.int32, 16) < 1,
                   jnp.float32(1.0), jnp.float32(0.0)).reshape(1, 16)

    @pl.loop(0, CHS)
    def _(r):
      for cc in range(D // 16):
        wbuf.at[pl.ds(r, 1), pl.ds(cc * 16, 16)][...] = zrow

    for z in ZOFFS:
      pltpu.sync_copy(wbuf, sh_acc.at[pl.ds(s * RPS + z, CHS)])

    @pl.loop(0, CHS)
    def _(r):
      wbuf.at[pl.ds(r, 1), pl.ds(DE, 16)][...] = e0

    pltpu.async_copy(dst_hbm.at[w, 0], dstv0, dsem0)
    pltpu.async_copy(dst_hbm.at[w, 1], dstv1, dsem1)

    plsc.subcore_barrier()

    pltpu.async_copy(ea_hbm.at[w, pl.ds(0, CHS)], ea0, esem0)
    pltpu.async_copy(ea_hbm.at[w, pl.ds(CHS, CHS)], ea1, esem1)

    @pl.loop(0, NCHS // 2)
    def _(jj):
      j = jj * 2
      pltpu.make_async_copy(ea_hbm.at[w, pl.ds(0, CHS)], ea0, esem0).wait()

      @pl.loop(0, CHS)
      def _(r):
        wbuf.at[pl.ds(r, 1), pl.ds(0, DE)][...] = ea0.at[pl.ds(r, 1), :][...]

      @pl.when(jj < NCHS // 2 - 1)
      def _():
        pltpu.async_copy(ea_hbm.at[w, pl.ds((j + 2) * CHS, CHS)], ea0,
                         esem0)

      pltpu.make_async_copy(dst_hbm.at[w, j], dstv0, dsem0).wait()
      pltpu.sync_copy(wbuf, sh_acc.at[dstv0], add=True)

      @pl.when(jj < NCHS // 2 - 1)
      def _():
        pltpu.async_copy(dst_hbm.at[w, j + 2], dstv0, dsem0)

      pltpu.make_async_copy(ea_hbm.at[w, pl.ds(0, CHS)], ea1, esem1).wait()

      @pl.loop(0, CHS)
      def _(r):
        wbuf.at[pl.ds(r, 1), pl.ds(0, DE)][...] = ea1.at[pl.ds(r, 1), :][...]

      @pl.when(jj < NCHS // 2 - 1)
      def _():
        pltpu.async_copy(ea_hbm.at[w, pl.ds((j + 3) * CHS, CHS)], ea1,
                         esem1)

      pltpu.make_async_copy(dst_hbm.at[w, j + 1], dstv1, dsem1).wait()
      pltpu.sync_copy(wbuf, sh_acc.at[dstv1], add=True)

      @pl.when(jj < NCHS // 2 - 1)
      def _():
        pltpu.async_copy(dst_hbm.at[w, j + 3], dstv1, dsem1)

    plsc.subcore_barrier()

    pltpu.sync_copy(sh_acc.at[pl.ds(s * RPS, RPS)],
                    sh_out.at[c, pl.ds(s * RPS, RPS)])

  return pl.kernel(body, mesh=_sc_mesh(), out_type=out_type,
                   scratch_types=scratch)


def _layer_block(x_ref, a_ref, sh_ref, w_ref, b_ref, g_ref, bt_ref):
  xb = x_ref[...]
  a = a_ref[...]
  sh = sh_ref[...]
  ab = a[0] + a[1] + xb
  sb = sh[0, :, 0:DE] + sh[1, :, 0:DE] + 1.0
  cnt = sh[0, :, DE:DE + 1] + sh[1, :, DE:DE + 1] + 1.0
  wi = w_ref[0:D, :]
  wj = w_ref[D:2 * D, :]
  we = w_ref[2 * D:2 * D + DE, :]
  agg = cnt * (_dot(xb, wi) + b_ref[...]) + _dot(ab, wj) + _dot(sb, we)
  h = jnp.maximum(agg, 0.0)
  h = h * (g_ref[...] * _BN) + bt_ref[...]
  return jnp.maximum(h, 0.0)


def _tc_layer1(x, a_part, sh_part, W, b, g, bt):
  def kern(x_ref, a_ref, sh_ref, w_ref, b_ref, g_ref, bt_ref, o_ref):
    o_ref[...] = _layer_block(x_ref, a_ref, sh_ref, w_ref, b_ref,
                              g_ref, bt_ref)

  return pl.pallas_call(
      kern,
      grid=(NBLK,),
      in_specs=[
          pl.BlockSpec((RB, D), lambda i: (i, 0)),
          pl.BlockSpec((NC, RB, D), lambda i: (0, i, 0)),
          pl.BlockSpec((NC, RB, D), lambda i: (0, i, 0)),
          pl.BlockSpec((2 * D + DE, H1), lambda i: (0, 0)),
          pl.BlockSpec((1, H1), lambda i: (0, 0)),
          pl.BlockSpec((1, H1), lambda i: (0, 0)),
          pl.BlockSpec((1, H1), lambda i: (0, 0)),
      ],
      out_specs=pl.BlockSpec((RB, H1), lambda i: (i, 0)),
      out_shape=jax.ShapeDtypeStruct((N, H1), jnp.float32),
  )(x, a_part, sh_part, W, b, g, bt)


def _tc_layer2(h1, a_part, sh_part, batch3, neighbor, W, b, g, bt,
               wf1a, wf1b, wf1c, bf1, wf2, bf2):
  def kern(x_ref, a_ref, sh_ref, batch_ref, nb_ref, w_ref, b_ref,
           g_ref, bt_ref, wf1a_ref, wf1b_ref, wf1c_ref, bf1_ref, wf2_ref,
           bf2_ref, o_ref, pool_scr, cnt_scr):
    i = pl.program_id(0)

    @pl.when(i == 0)
    def _():
      pool_scr[...] = jnp.zeros_like(pool_scr)
      cnt_scr[...] = jnp.zeros_like(cnt_scr)

    h2 = _layer_block(x_ref, a_ref, sh_ref, w_ref, b_ref, g_ref, bt_ref)
    bvec = batch_ref[0, 0, :]
    onehot = (bvec[:, None] ==
              lax.broadcasted_iota(jnp.int32, (RB, G), 1)).astype(jnp.float32)
    pool_scr[...] += _dot_t(onehot, h2)
    cnt_scr[...] += _dot_t(onehot, jnp.ones((RB, 8), jnp.float32))

    @pl.when(i == NBLK - 1)
    def _():
      pooled = pool_scr[...]
      counts = cnt_scr[...][:, 0:1] * (1.0 / 40.0)
      hid = (_dot(pooled, wf1a_ref[...]) + counts * wf1b_ref[...] +
             _dot(nb_ref[...], wf1c_ref[...]) + bf1_ref[...])
      hid = jnp.maximum(hid, 0.0)
      o_ref[...] = _dot(hid, wf2_ref[...]) + bf2_ref[...]

  return pl.pallas_call(
      kern,
      grid=(NBLK,),
      in_specs=[
          pl.BlockSpec((RB, D), lambda i: (i, 0)),
          pl.BlockSpec((NC, RB, D), lambda i: (0, i, 0)),
          pl.BlockSpec((NC, RB, D), lambda i: (0, i, 0)),
          pl.BlockSpec((1, 1, RB), lambda i: (i, 0, 0)),
          pl.BlockSpec((G, D), lambda i: (0, 0)),
          pl.BlockSpec((2 * D + DE, H1), lambda i: (0, 0)),
          pl.BlockSpec((1, H1), lambda i: (0, 0)),
          pl.BlockSpec((1, H1), lambda i: (0, 0)),
          pl.BlockSpec((1, H1), lambda i: (0, 0)),
          pl.BlockSpec((D, MLP_DIM), lambda i: (0, 0)),
          pl.BlockSpec((1, MLP_DIM), lambda i: (0, 0)),
          pl.BlockSpec((D, MLP_DIM), lambda i: (0, 0)),
          pl.BlockSpec((1, MLP_DIM), lambda i: (0, 0)),
          pl.BlockSpec((MLP_DIM, NCLS), lambda i: (0, 0)),
          pl.BlockSpec((1, NCLS), lambda i: (0, 0)),
      ],
      out_specs=pl.BlockSpec((G, NCLS), lambda i: (0, 0)),
      out_shape=jax.ShapeDtypeStruct((G, NCLS), jnp.float32),
      scratch_shapes=[pltpu.VMEM((G, D), jnp.float32),
                      pltpu.VMEM((G, 8), jnp.float32)],
  )(h1, a_part, sh_part, batch3, neighbor, W, b, g, bt,
    wf1a, wf1b, wf1c, bf1, wf2, bf2)


def kernel(x, edge_index, edge_attr, batch, neighbor, W1, b1, g1, bt1,
           W2, b2, g2, bt2, Wf1, bf1, Wf2, bf2):
  pad = EPAD - E
  src = jnp.concatenate([edge_index[0].astype(jnp.int32),
                         jnp.zeros((pad,), jnp.int32)])
  dst = jnp.concatenate([edge_index[1].astype(jnp.int32),
                         jnp.full((pad,), N, jnp.int32)])
  dst3 = dst.reshape(NW, NCHS, CHS)
  ea = jnp.concatenate([edge_attr, jnp.zeros((pad, DE), jnp.float32)],
                       axis=0).reshape(NW, EPW, DE)

  a1 = _make_sc_edge_agg()(x, src, dst)
  sh = _make_sc_edge_sh()(dst3, ea)
  h1 = _tc_layer1(x, a1, sh, W1, b1.reshape(1, -1),
                  g1.reshape(1, -1), bt1.reshape(1, -1))
  a2 = _make_sc_edge_agg()(h1, src, dst)
  batch3 = batch.astype(jnp.int32).reshape(NBLK, 1, RB)
  out = _tc_layer2(h1, a2, sh, batch3, neighbor, W2,
                   b2.reshape(1, -1), g2.reshape(1, -1), bt2.reshape(1, -1),
                   Wf1[0:D], Wf1[D:D + 1], Wf1[D + 1:], bf1.reshape(1, -1),
                   Wf2, bf2.reshape(1, -1))
  return out


# speedup vs baseline: 1.1483x; 1.1355x over previous
import functools
import jax
import jax.numpy as jnp
from jax import lax
from jax.experimental import pallas as pl
from jax.experimental.pallas import tpu as pltpu
from jax.experimental.pallas import tpu_sc as plsc

N = 10000
E = 320000
D = 128
DE = 16
G = 64
H1 = 128
MLP_DIM = 256
NCLS = 4

NC = 2
NS = 16
NW = NC * NS
CHUNK = 128
EPW = -(-E // (NW * CHUNK)) * CHUNK
EPAD = EPW * NW
NCH = EPW // CHUNK
NACC = 10112
RPS = NACC // NS
ZOFF = (0, 128, 256, 384, 504)

RB = 1000
NBLK = N // RB

_BN = float(1.0 / (1.0 + 1e-5) ** 0.5)


def _dot(a, b):
  return lax.dot_general(a, b, (((1,), (0,)), ((), ())),
                         precision=lax.Precision.HIGHEST,
                         preferred_element_type=jnp.float32)


def _dot_t(a, b):
  return lax.dot_general(a, b, (((0,), (0,)), ((), ())),
                         precision=lax.Precision.HIGHEST,
                         preferred_element_type=jnp.float32)


def _sc_mesh():
  return plsc.VectorSubcoreMesh(core_axis_name="c", subcore_axis_name="s")


def _make_sc_edge_agg():
  out_type = jax.ShapeDtypeStruct((NC, NACC, D), jnp.float32)
  scratch = [
      pltpu.VMEM((CHUNK,), jnp.int32),
      pltpu.VMEM((CHUNK,), jnp.int32),
      pltpu.VMEM((CHUNK, D), jnp.float32),
      pltpu.VMEM_SHARED((NACC, D), jnp.float32),
      pltpu.SemaphoreType.DMA,
  ]

  def body(table_hbm, src_hbm, dst_hbm, a_out, srcv, dstv, rows, a_sh, sem):
    c = lax.axis_index("c")
    s = lax.axis_index("s")
    w = c * NS + s

    zrow = jnp.zeros((1, 16), jnp.float32)

    @pl.loop(0, CHUNK)
    def _(r):
      for cc in range(D // 16):
        rows.at[pl.ds(r, 1), pl.ds(cc * 16, 16)][...] = zrow

    for z in ZOFF:
      pltpu.sync_copy(rows, a_sh.at[pl.ds(s * RPS + z, CHUNK)])

    plsc.subcore_barrier()

    base = w * EPW

    @pl.loop(0, NCH)
    def _(j):
      off = base + j * CHUNK
      pltpu.sync_copy(src_hbm.at[pl.ds(off, CHUNK)], srcv)
      pltpu.sync_copy(dst_hbm.at[pl.ds(off, CHUNK)], dstv)
      pltpu.async_copy(table_hbm.at[srcv], rows, sem).wait()
      pltpu.sync_copy(rows, a_sh.at[dstv], add=True)

    plsc.subcore_barrier()

    pltpu.sync_copy(a_sh.at[pl.ds(s * RPS, RPS)],
                    a_out.at[c, pl.ds(s * RPS, RPS)])

  return pl.kernel(body, mesh=_sc_mesh(), out_type=out_type,
                   scratch_types=scratch)


def _make_sc_edge_sh():
  out_type = jax.ShapeDtypeStruct((NC, NACC, D), jnp.float32)
  scratch = [
      pltpu.VMEM((CHUNK,), jnp.int32),
      pltpu.VMEM((CHUNK, DE), jnp.float32),
      pltpu.VMEM((CHUNK, D), jnp.float32),
      pltpu.VMEM_SHARED((NACC, D), jnp.float32),
      pltpu.SemaphoreType.DMA,
  ]

  def body(dst_hbm, ea_hbm, sh_out, dstv, eav, wbuf, sh_acc, sem):
    c = lax.axis_index("c")
    s = lax.axis_index("s")
    w = c * NS + s

    zrow = jnp.zeros((1, 16), jnp.float32)
    e0 = jnp.where(lax.iota(jnp.int32, 16) < 1,
                   jnp.float32(1.0), jnp.float32(0.0)).reshape(1, 16)

    @pl.loop(0, CHUNK)
    def _(r):
      for cc in range(D // 16):
        wbuf.at[pl.ds(r, 1), pl.ds(cc * 16, 16)][...] = zrow

    for z in ZOFF:
      pltpu.sync_copy(wbuf, sh_acc.at[pl.ds(s * RPS + z, CHUNK)])

    @pl.loop(0, CHUNK)
    def _(r):
      wbuf.at[pl.ds(r, 1), pl.ds(DE, 16)][...] = e0

    plsc.subcore_barrier()

    base = w * EPW

    @pl.loop(0, NCH)
    def _(j):
      off = base + j * CHUNK
      pltpu.sync_copy(dst_hbm.at[pl.ds(off, CHUNK)], dstv)
      pltpu.sync_copy(ea_hbm.at[pl.ds(off, CHUNK)], eav)

      @pl.loop(0, CHUNK)
      def _(r):
        wbuf.at[pl.ds(r, 1), pl.ds(0, DE)][...] = eav.at[pl.ds(r, 1), :][...]

      pltpu.sync_copy(wbuf, sh_acc.at[dstv], add=True)

    plsc.subcore_barrier()

    pltpu.sync_copy(sh_acc.at[pl.ds(s * RPS, RPS)],
                    sh_out.at[c, pl.ds(s * RPS, RPS)])

  return pl.kernel(body, mesh=_sc_mesh(), out_type=out_type,
                   scratch_types=scratch)


def _layer_block(x_ref, a_ref, sh_ref, w_ref, b_ref, g_ref, bt_ref):
  xb = x_ref[...]
  a = a_ref[...]
  sh = sh_ref[...]
  ab = a[0] + a[1] + xb
  sb = sh[0, :, 0:DE] + sh[1, :, 0:DE] + 1.0
  cnt = sh[0, :, DE:DE + 1] + sh[1, :, DE:DE + 1] + 1.0
  wi = w_ref[0:D, :]
  wj = w_ref[D:2 * D, :]
  we = w_ref[2 * D:2 * D + DE, :]
  agg = cnt * (_dot(xb, wi) + b_ref[...]) + _dot(ab, wj) + _dot(sb, we)
  h = jnp.maximum(agg, 0.0)
  h = h * (g_ref[...] * _BN) + bt_ref[...]
  return jnp.maximum(h, 0.0)


def _tc_layer1(x, a_part, sh_part, W, b, g, bt):
  def kern(x_ref, a_ref, sh_ref, w_ref, b_ref, g_ref, bt_ref, o_ref):
    o_ref[...] = _layer_block(x_ref, a_ref, sh_ref, w_ref, b_ref,
                              g_ref, bt_ref)

  return pl.pallas_call(
      kern,
      grid=(NBLK,),
      in_specs=[
          pl.BlockSpec((RB, D), lambda i: (i, 0)),
          pl.BlockSpec((NC, RB, D), lambda i: (0, i, 0)),
          pl.BlockSpec((NC, RB, D), lambda i: (0, i, 0)),
          pl.BlockSpec((2 * D + DE, H1), lambda i: (0, 0)),
          pl.BlockSpec((1, H1), lambda i: (0, 0)),
          pl.BlockSpec((1, H1), lambda i: (0, 0)),
          pl.BlockSpec((1, H1), lambda i: (0, 0)),
      ],
      out_specs=pl.BlockSpec((RB, H1), lambda i: (i, 0)),
      out_shape=jax.ShapeDtypeStruct((N, H1), jnp.float32),
  )(x, a_part, sh_part, W, b, g, bt)


def _tc_layer2(h1, a_part, sh_part, batch3, neighbor, W, b, g, bt,
               wf1a, wf1b, wf1c, bf1, wf2, bf2):
  def kern(x_ref, a_ref, sh_ref, batch_ref, nb_ref, w_ref, b_ref,
           g_ref, bt_ref, wf1a_ref, wf1b_ref, wf1c_ref, bf1_ref, wf2_ref,
           bf2_ref, o_ref, pool_scr, cnt_scr):
    i = pl.program_id(0)

    @pl.when(i == 0)
    def _():
      pool_scr[...] = jnp.zeros_like(pool_scr)
      cnt_scr[...] = jnp.zeros_like(cnt_scr)

    h2 = _layer_block(x_ref, a_ref, sh_ref, w_ref, b_ref, g_ref, bt_ref)
    bvec = batch_ref[0, 0, :]
    onehot = (bvec[:, None] ==
              lax.broadcasted_iota(jnp.int32, (RB, G), 1)).astype(jnp.float32)
    pool_scr[...] += _dot_t(onehot, h2)
    cnt_scr[...] += _dot_t(onehot, jnp.ones((RB, 8), jnp.float32))

    @pl.when(i == NBLK - 1)
    def _():
      pooled = pool_scr[...]
      counts = cnt_scr[...][:, 0:1] * (1.0 / 40.0)
      hid = (_dot(pooled, wf1a_ref[...]) + counts * wf1b_ref[...] +
             _dot(nb_ref[...], wf1c_ref[...]) + bf1_ref[...])
      hid = jnp.maximum(hid, 0.0)
      o_ref[...] = _dot(hid, wf2_ref[...]) + bf2_ref[...]

  return pl.pallas_call(
      kern,
      grid=(NBLK,),
      in_specs=[
          pl.BlockSpec((RB, D), lambda i: (i, 0)),
          pl.BlockSpec((NC, RB, D), lambda i: (0, i, 0)),
          pl.BlockSpec((NC, RB, D), lambda i: (0, i, 0)),
          pl.BlockSpec((1, 1, RB), lambda i: (i, 0, 0)),
          pl.BlockSpec((G, D), lambda i: (0, 0)),
          pl.BlockSpec((2 * D + DE, H1), lambda i: (0, 0)),
          pl.BlockSpec((1, H1), lambda i: (0, 0)),
          pl.BlockSpec((1, H1), lambda i: (0, 0)),
          pl.BlockSpec((1, H1), lambda i: (0, 0)),
          pl.BlockSpec((D, MLP_DIM), lambda i: (0, 0)),
          pl.BlockSpec((1, MLP_DIM), lambda i: (0, 0)),
          pl.BlockSpec((D, MLP_DIM), lambda i: (0, 0)),
          pl.BlockSpec((1, MLP_DIM), lambda i: (0, 0)),
          pl.BlockSpec((MLP_DIM, NCLS), lambda i: (0, 0)),
          pl.BlockSpec((1, NCLS), lambda i: (0, 0)),
      ],
      out_specs=pl.BlockSpec((G, NCLS), lambda i: (0, 0)),
      out_shape=jax.ShapeDtypeStruct((G, NCLS), jnp.float32),
      scratch_shapes=[pltpu.VMEM((G, D), jnp.float32),
                      pltpu.VMEM((G, 8), jnp.float32)],
  )(h1, a_part, sh_part, batch3, neighbor, W, b, g, bt,
    wf1a, wf1b, wf1c, bf1, wf2, bf2)


def kernel(x, edge_index, edge_attr, batch, neighbor, W1, b1, g1, bt1,
           W2, b2, g2, bt2, Wf1, bf1, Wf2, bf2):
  pad = EPAD - E
  src = jnp.concatenate([edge_index[0].astype(jnp.int32),
                         jnp.zeros((pad,), jnp.int32)])
  dst = jnp.concatenate([edge_index[1].astype(jnp.int32),
                         jnp.full((pad,), N, jnp.int32)])
  ea = jnp.concatenate([edge_attr, jnp.zeros((pad, DE), jnp.float32)], axis=0)

  a1 = _make_sc_edge_agg()(x, src, dst)
  sh = _make_sc_edge_sh()(dst, ea)
  h1 = _tc_layer1(x, a1, sh, W1, b1.reshape(1, -1),
                  g1.reshape(1, -1), bt1.reshape(1, -1))
  a2 = _make_sc_edge_agg()(h1, src, dst)
  batch3 = batch.astype(jnp.int32).reshape(NBLK, 1, RB)
  out = _tc_layer2(h1, a2, sh, batch3, neighbor, W2,
                   b2.reshape(1, -1), g2.reshape(1, -1), bt2.reshape(1, -1),
                   Wf1[0:D], Wf1[D:D + 1], Wf1[D + 1:], bf1.reshape(1, -1),
                   Wf2, bf2.reshape(1, -1))
  return out
